# baseline (speedup 1.0000x reference)
import functools

import jax
import jax.numpy as jnp
from jax import lax
from jax.experimental import pallas as pl
from jax.experimental.pallas import tpu as pltpu
from jax.experimental.pallas import tpu_sc as plsc

_BATCH = 4096
_SEQ = 200
_VOCAB = 100000

_NC = 2
_NS = 16
_NW = _NC * _NS
_COLS_PER = _BATCH // _NW
_LANES = 16
_VOCAB_PAD = 100096
_VECS_PER_ROW = _COLS_PER // _LANES


_CHUNK_ROWS = 40
_N_CHUNKS = _SEQ // _CHUNK_ROWS


_HALF = _VOCAB_PAD // 2
_UPPER = _VOCAB_PAD - _HALF
_TAB_SLICE = _UPPER // _NS
_LAST_START = _VOCAB - _TAB_SLICE


def _lookup_body(tok_hbm, tab_hbm, out_hbm, spm_tab, tab_v, slice_v,
                 i0, i1, i2, sem_lo, sem_tab, si0, si1, si2, so0, so1, so2):
    bufs = (i0, i1, i2)
    sin = (si0, si1, si2)
    sout = (so0, so1, so2)
    sid = lax.axis_index("s")
    wid = sid * _NC + lax.axis_index("c")
    col0 = wid * _COLS_PER

    def hbm_slice(c):
        return (pl.ds(c * _CHUNK_ROWS, _CHUNK_ROWS), pl.ds(col0, _COLS_PER))

    def start_in(c):
        return pltpu.async_copy(
            tok_hbm.at[hbm_slice(c)], bufs[c % 3], sin[c % 3])

    def start_out(c):
        return pltpu.async_copy(
            bufs[c % 3], out_hbm.at[hbm_slice(c)], sout[c % 3])

    in_dmas = {c: start_in(c) for c in range(3)}
    out_dmas = {}

    lo_dma = pltpu.async_copy(
        tab_hbm.at[pl.ds(0, _HALF)], tab_v.at[pl.ds(0, _HALF)], sem_lo)

    start = pl.multiple_of(
        jnp.where(sid == _NS - 1, _LAST_START, _HALF + sid * _TAB_SLICE), 8)
    pltpu.async_copy(
        tab_hbm.at[pl.ds(start, _TAB_SLICE)], slice_v, sem_tab).wait()
    pltpu.async_copy(
        slice_v, spm_tab.at[pl.ds(start, _TAB_SLICE)], sem_tab).wait()
    plsc.subcore_barrier()
    up_dma = pltpu.async_copy(
        spm_tab.at[pl.ds(_HALF, _UPPER)], tab_v.at[pl.ds(_HALF, _UPPER)],
        sem_tab)
    lo_dma.wait()
    up_dma.wait()

    for c in range(_N_CHUNKS):
        if 1 <= c and c + 2 < _N_CHUNKS:
            out_dmas[c - 1].wait()
            in_dmas[c + 2] = start_in(c + 2)
        in_dmas[c].wait()
        buf = bufs[c % 3]

        @plsc.parallel_loop(0, _CHUNK_ROWS, 1, unroll=4)
        def _gather_row(r, buf=buf):
            for j in range(_VECS_PER_ROW):
                idx = buf[r, pl.ds(j * _LANES, _LANES)]
                buf[r, pl.ds(j * _LANES, _LANES)] = plsc.load_gather(tab_v, [idx])

        out_dmas[c] = start_out(c)
    for c in range(_N_CHUNKS):
        if c not in (0, 1):
            out_dmas[c].wait()


@jax.jit
def kernel(tokens, vocab_map):
    run = functools.partial(
        pl.kernel,
        mesh=plsc.VectorSubcoreMesh(core_axis_name="c", subcore_axis_name="s"),
        out_type=jax.ShapeDtypeStruct((_SEQ, _BATCH), jnp.int32),
        scratch_types=[
            pltpu.VMEM_SHARED((_VOCAB_PAD,), jnp.int32),
            pltpu.VMEM((_VOCAB_PAD,), jnp.int32),
            pltpu.VMEM((_TAB_SLICE,), jnp.int32),
            pltpu.VMEM((_CHUNK_ROWS, _COLS_PER), jnp.int32),
            pltpu.VMEM((_CHUNK_ROWS, _COLS_PER), jnp.int32),
            pltpu.VMEM((_CHUNK_ROWS, _COLS_PER), jnp.int32),
        ] + [pltpu.SemaphoreType.DMA] * 8,
        compiler_params=pltpu.CompilerParams(
            needs_layout_passes=False, use_tc_tiling_on_sc=True
        ),
    )(_lookup_body)
    return run(tokens.T, vocab_map).T


# speedup vs baseline: 1.0608x; 1.0608x over previous
import functools

import jax
import jax.numpy as jnp
from jax import lax
from jax.experimental import pallas as pl
from jax.experimental.pallas import tpu as pltpu
from jax.experimental.pallas import tpu_sc as plsc

_BATCH = 4096
_SEQ = 200
_VOCAB = 100000

_NC = 2
_NS = 16
_NW = _NC * _NS
_COLS_PER = _BATCH // _NW
_LANES = 16
_VOCAB_PAD = 100096
_VECS_PER_ROW = _COLS_PER // _LANES


_CHUNK_ROWS = 40
_N_CHUNKS = _SEQ // _CHUNK_ROWS


_TAB_SLICE = _VOCAB_PAD // _NS
_LAST_START = _VOCAB - _TAB_SLICE


def _lookup_body(tok_hbm, tab_hbm, out_hbm, spm_tab, tab_v, slice_v,
                 i0, i1, i2, sem_tab, si0, si1, si2, so0, so1, so2):
    bufs = (i0, i1, i2)
    sin = (si0, si1, si2)
    sout = (so0, so1, so2)
    sid = lax.axis_index("s")
    wid = sid * _NC + lax.axis_index("c")
    col0 = wid * _COLS_PER

    def hbm_slice(c):
        return (pl.ds(c * _CHUNK_ROWS, _CHUNK_ROWS), pl.ds(col0, _COLS_PER))

    def start_in(c):
        return pltpu.async_copy(
            tok_hbm.at[hbm_slice(c)], bufs[c % 3], sin[c % 3])

    def start_out(c):
        return pltpu.async_copy(
            bufs[c % 3], out_hbm.at[hbm_slice(c)], sout[c % 3])

    in_dmas = {c: start_in(c) for c in range(3)}
    out_dmas = {}

    start = pl.multiple_of(
        jnp.where(sid == _NS - 1, _LAST_START, sid * _TAB_SLICE), 8)
    pltpu.async_copy(
        tab_hbm.at[pl.ds(start, _TAB_SLICE)], slice_v, sem_tab).wait()
    pltpu.async_copy(
        slice_v, spm_tab.at[pl.ds(start, _TAB_SLICE)], sem_tab).wait()
    plsc.subcore_barrier()
    pltpu.sync_copy(spm_tab, tab_v)

    for c in range(_N_CHUNKS):
        if 1 <= c and c + 2 < _N_CHUNKS:
            out_dmas[c - 1].wait()
            in_dmas[c + 2] = start_in(c + 2)
        in_dmas[c].wait()
        buf = bufs[c % 3]

        @plsc.parallel_loop(0, _CHUNK_ROWS, 1, unroll=4)
        def _gather_row(r, buf=buf):
            for j in range(_VECS_PER_ROW):
                idx = buf[r, pl.ds(j * _LANES, _LANES)]
                buf[r, pl.ds(j * _LANES, _LANES)] = plsc.load_gather(tab_v, [idx])

        out_dmas[c] = start_out(c)
    for c in range(_N_CHUNKS):
        if c not in (0, 1):
            out_dmas[c].wait()


@jax.jit
def kernel(tokens, vocab_map):
    run = functools.partial(
        pl.kernel,
        mesh=plsc.VectorSubcoreMesh(core_axis_name="c", subcore_axis_name="s"),
        out_type=jax.ShapeDtypeStruct((_SEQ, _BATCH), jnp.int32),
        scratch_types=[
            pltpu.VMEM_SHARED((_VOCAB_PAD,), jnp.int32),
            pltpu.VMEM((_VOCAB_PAD,), jnp.int32),
            pltpu.VMEM((_TAB_SLICE,), jnp.int32),
            pltpu.VMEM((_CHUNK_ROWS, _COLS_PER), jnp.int32),
            pltpu.VMEM((_CHUNK_ROWS, _COLS_PER), jnp.int32),
            pltpu.VMEM((_CHUNK_ROWS, _COLS_PER), jnp.int32),
            pltpu.SemaphoreType.DMA,
            pltpu.SemaphoreType.DMA,
            pltpu.SemaphoreType.DMA,
            pltpu.SemaphoreType.DMA,
            pltpu.SemaphoreType.DMA,
            pltpu.SemaphoreType.DMA,
            pltpu.SemaphoreType.DMA,
        ],
        compiler_params=pltpu.CompilerParams(
            needs_layout_passes=False,
            use_tc_tiling_on_sc=True,
            skip_device_barrier=True,
            disable_bounds_checks=True,
        ),
    )(_lookup_body)
    return run(tokens.T, vocab_map).T
